# Initial kernel scaffold; baseline (speedup 1.0000x reference)
import jax
import jax.numpy as jnp
from jax.experimental import pallas as pl


def kernel(x, Wr1, br1, Wr2, br2, Wr3, br3, We1, be1, We2, be2, We3, be3):
    raise NotImplementedError("write your pallas kernel here")


# speedup vs baseline: 1.0802x; 1.0802x over previous
import functools

import jax
import jax.numpy as jnp
from jax import lax
from jax.experimental import pallas as pl
from jax.experimental.pallas import tpu as pltpu

E = 8
TOPK = 2
IN = 1024
RH = 512
RH2 = 256
H1 = 1024
H2 = 512
NC = 256
N = 2048
NB = 256
EP = 128
NT = N // NB


def _router_body(x_ref, wr1_ref, br1_ref, wr2_ref, br2_ref, wr3_ref, br3_ref,
                 probs_ref, w2_ref):
    x = x_ref[...]
    h = jnp.maximum(
        jnp.dot(x, wr1_ref[...], preferred_element_type=jnp.float32)
        + br1_ref[...], 0.0)
    h = jnp.maximum(
        jnp.dot(h, wr2_ref[...], preferred_element_type=jnp.float32)
        + br2_ref[...], 0.0)
    s = jnp.dot(h, wr3_ref[...], preferred_element_type=jnp.float32) \
        + br3_ref[...]
    lane = lax.broadcasted_iota(jnp.int32, (NB, EP), 1)
    s = jnp.where(lane < E, s, -1e30)
    m = jnp.max(s, axis=1, keepdims=True)
    p = jnp.exp(s - m)
    probs = p / jnp.sum(p, axis=1, keepdims=True)
    probs_ref[...] = probs
    v1 = jnp.max(probs, axis=1, keepdims=True)
    i1 = jnp.min(jnp.where(probs == v1, lane, EP), axis=1, keepdims=True)
    pm = jnp.where(lane == i1, -1.0, probs)
    v2 = jnp.max(pm, axis=1, keepdims=True)
    i2 = jnp.min(jnp.where(pm == v2, lane, EP), axis=1, keepdims=True)
    w2_ref[...] = jnp.where((lane == i1) | (lane == i2), probs * 0.5, 0.0)


def _expert_body(w2_ref, x_ref, we1_ref, be1_ref, we2_ref, be2_ref,
                 we3_ref, be3_ref, out_ref):
    e = pl.program_id(0)
    t = pl.program_id(1)
    x = x_ref[...]
    h1 = jnp.maximum(
        jnp.dot(x, we1_ref[0], preferred_element_type=jnp.float32)
        + be1_ref[0], 0.0)
    h2 = jnp.maximum(
        jnp.dot(h1, we2_ref[0], preferred_element_type=jnp.float32)
        + be2_ref[0], 0.0)
    eo = jnp.dot(h2, we3_ref[0], preferred_element_type=jnp.float32) \
        + be3_ref[0]
    lane = lax.broadcasted_iota(jnp.int32, (NB, EP), 1)
    w_col = jnp.sum(jnp.where(lane == e, w2_ref[...], 0.0), axis=1,
                    keepdims=True)
    contrib = w_col * eo

    @pl.when(e == 0)
    def _init():
        out_ref[pl.ds(t * NB, NB), :] = contrib

    @pl.when(e != 0)
    def _acc():
        out_ref[pl.ds(t * NB, NB), :] += contrib


@jax.jit
def kernel(x, Wr1, br1, Wr2, br2, Wr3, br3, We1, be1, We2, be2, We3, be3):
    wr3p = jnp.pad(Wr3, ((0, 0), (0, EP - E)))
    br3p = jnp.pad(br3, (0, EP - E)).reshape(1, EP)

    probs_full, w2 = pl.pallas_call(
        _router_body,
        grid=(NT,),
        in_specs=[
            pl.BlockSpec((NB, IN), lambda t: (t, 0)),
            pl.BlockSpec((IN, RH), lambda t: (0, 0)),
            pl.BlockSpec((1, RH), lambda t: (0, 0)),
            pl.BlockSpec((RH, RH2), lambda t: (0, 0)),
            pl.BlockSpec((1, RH2), lambda t: (0, 0)),
            pl.BlockSpec((RH2, EP), lambda t: (0, 0)),
            pl.BlockSpec((1, EP), lambda t: (0, 0)),
        ],
        out_specs=[
            pl.BlockSpec((NB, EP), lambda t: (t, 0)),
            pl.BlockSpec((NB, EP), lambda t: (t, 0)),
        ],
        out_shape=[
            jax.ShapeDtypeStruct((N, EP), jnp.float32),
            jax.ShapeDtypeStruct((N, EP), jnp.float32),
        ],
    )(x, Wr1, br1.reshape(1, RH), Wr2, br2.reshape(1, RH2), wr3p, br3p)

    out = pl.pallas_call(
        _expert_body,
        grid=(E, NT),
        in_specs=[
            pl.BlockSpec((NB, EP), lambda e, t: (t, 0)),
            pl.BlockSpec((NB, IN), lambda e, t: (t, 0)),
            pl.BlockSpec((1, IN, H1), lambda e, t: (e, 0, 0)),
            pl.BlockSpec((1, 1, H1), lambda e, t: (e, 0, 0)),
            pl.BlockSpec((1, H1, H2), lambda e, t: (e, 0, 0)),
            pl.BlockSpec((1, 1, H2), lambda e, t: (e, 0, 0)),
            pl.BlockSpec((1, H2, NC), lambda e, t: (e, 0, 0)),
            pl.BlockSpec((1, 1, NC), lambda e, t: (e, 0, 0)),
        ],
        out_specs=pl.BlockSpec((N, NC), lambda e, t: (0, 0)),
        out_shape=jax.ShapeDtypeStruct((N, NC), jnp.float32),
        compiler_params=pltpu.CompilerParams(
            dimension_semantics=("arbitrary", "arbitrary")),
    )(w2, x, We1, be1.reshape(E, 1, H1), We2, be2.reshape(E, 1, H2),
      We3, be3.reshape(E, 1, NC))

    return out, probs_full[:, :E]
